# baseline (speedup 1.0000x reference)
import jax
import jax.numpy as jnp
from jax.experimental import pallas as pl
from jax.experimental.pallas import tpu as pltpu

CTX = 256
PRED = 64
LAGS = [1, 2, 3, 4, 5, 6, 7, 24, 48]
HIST = CTX + max(LAGS)
DM = 1024
NH = 16
DH = DM // NH
NE = 4
ND = 4
DFF = 4096
SUB = CTX + PRED

_F32 = jnp.float32


def _ln(x):
    m = jnp.mean(x, axis=-1, keepdims=True)
    v = jnp.mean((x - m) ** 2, axis=-1, keepdims=True)
    return (x - m) / jnp.sqrt(v + 1e-5)


def _mm(a, b):
    return jax.lax.dot_general(a, b, (((1,), (0,)), ((), ())),
                               preferred_element_type=_F32)


def _mm_t(a, b):
    return jax.lax.dot_general(a, b, (((1,), (1,)), ((), ())),
                               preferred_element_type=_F32)


_BF16 = jnp.bfloat16


def _mmb(a, b):
    return _mm(a.astype(_BF16), b.astype(_BF16))


def _mmb_t(a, b):
    return _mm_t(a.astype(_BF16), b.astype(_BF16))


def _softmax(s):
    s = s - jnp.max(s, axis=-1, keepdims=True)
    e = jnp.exp(s)
    return e / jnp.sum(e, axis=-1, keepdims=True)


def _embed_body(t_ref, w_ref, o_ref):
    o_ref[...] = _mmb(t_ref[...], w_ref[...]).astype(_BF16)


def _embed(ti_in, w_pad):
    m = ti_in.shape[0]
    blk = 2048
    return pl.pallas_call(
        _embed_body,
        grid=(m // blk,),
        in_specs=[pl.BlockSpec((blk, 128), lambda i: (i, 0)),
                  pl.BlockSpec((128, DM), lambda i: (0, 0))],
        out_specs=pl.BlockSpec((blk, DM), lambda i: (i, 0)),
        out_shape=jax.ShapeDtypeStruct((m, DM), _BF16),
        compiler_params=pltpu.CompilerParams(
            dimension_semantics=("parallel",)),
    )(ti_in, w_pad)


G_ENC = 2


def _seg_ones(rows, groups, group):
    rr = jax.lax.broadcasted_iota(jnp.int32, (rows, groups), 0)
    cc = jax.lax.broadcasted_iota(jnp.int32, (rows, groups), 1)
    return (rr // group == cc).astype(_F32)


def _expand_ones(groups, cols, group):
    rr = jax.lax.broadcasted_iota(jnp.int32, (groups, cols), 0)
    cc = jax.lax.broadcasted_iota(jnp.int32, (groups, cols), 1)
    return (cc // group == rr).astype(_F32)


def _enc_layer_body(x_ref, wqk_ref, wv_ref, wo_ref, w1_ref, w2_ref,
                    o_ref, s_ref):
    rows = G_ENC * CTX
    x = x_ref[...].reshape(rows, DM).astype(_F32)
    h = _ln(x)
    hb = h.astype(_BF16)
    qk = _mm(hb, wqk_ref[...])
    v = _mm(hb, wv_ref[...])
    nrm2 = _mm(qk * qk, _seg_ones(DM, NH, DH))
    rinv = 1.0 / (jnp.sqrt(nrm2) + 1e-8)
    k = qk * _mm(rinv, _expand_ones(NH, DM, DH))
    qb = (qk * 0.125).astype(_BF16)
    kb = k.astype(_BF16)
    for g in range(G_ENC):
        rs = slice(g * CTX, (g + 1) * CTX)
        for hh in range(NH):
            cs = slice(hh * DH, (hh + 1) * DH)
            s_ref[rs, hh * CTX:(hh + 1) * CTX] = _mm_t(qb[rs, cs], kb[rs, cs])
    s = s_ref[...]
    m = jnp.max(s, axis=-1, keepdims=True)
    e = jnp.exp(s - m)
    eb = e.astype(_BF16)
    denom = _mm(eb, _seg_ones(NH * CTX, NH, CTX).astype(_BF16))
    dinv = 1.0 / denom
    vb = v.astype(_BF16)
    row_blocks = []
    for g in range(G_ENC):
        rs = slice(g * CTX, (g + 1) * CTX)
        cols = []
        for hh in range(NH):
            o_h = _mm(eb[rs, hh * CTX:(hh + 1) * CTX],
                      vb[rs, hh * DH:(hh + 1) * DH])
            cols.append(o_h * dinv[rs, hh:hh + 1])
        row_blocks.append(jnp.concatenate(cols, axis=1))
    o = jnp.concatenate(row_blocks, axis=0)
    xa = x + _mm(o.astype(_BF16), wo_ref[...])
    h2b = _ln(xa).astype(_BF16)
    half = DFF // 2
    t1 = jax.nn.gelu(_mm(h2b, w1_ref[:, :half]))
    o1 = _mm(t1.astype(_BF16), w2_ref[:half, :])
    t2 = jax.nn.gelu(_mm(h2b, w1_ref[:, half:]))
    o2 = _mm(t2.astype(_BF16), w2_ref[half:, :])
    o_ref[...] = (xa + o1 + o2).astype(_BF16).reshape(G_ENC, CTX, DM)


def _enc_layer(x, wqk, wv, wo, w1, w2):
    b = x.shape[0]
    return pl.pallas_call(
        _enc_layer_body,
        grid=(b // G_ENC,),
        in_specs=[pl.BlockSpec((G_ENC, CTX, DM), lambda i: (i, 0, 0)),
                  pl.BlockSpec((DM, DM), lambda i: (0, 0)),
                  pl.BlockSpec((DM, DM), lambda i: (0, 0)),
                  pl.BlockSpec((DM, DM), lambda i: (0, 0)),
                  pl.BlockSpec((DM, DFF), lambda i: (0, 0)),
                  pl.BlockSpec((DFF, DM), lambda i: (0, 0))],
        out_specs=pl.BlockSpec((G_ENC, CTX, DM), lambda i: (i, 0, 0)),
        out_shape=jax.ShapeDtypeStruct(x.shape, _BF16),
        scratch_shapes=[pltpu.VMEM((G_ENC * CTX, NH * CTX), _F32)],
        compiler_params=pltpu.CompilerParams(
            dimension_semantics=("parallel",)),
    )(x, wqk, wv, wo, w1, w2)


def _ff_body(x_ref, w1_ref, w2_ref, o_ref):
    x = x_ref[0]
    h = _ln(x)
    t = jax.nn.gelu(_mm(h.astype(_BF16), w1_ref[...]))
    o_ref[0] = x + _mmb(t, w2_ref[...])


def _ff(x, w1, w2, rows):
    b = x.shape[0]
    return pl.pallas_call(
        _ff_body,
        grid=(b,),
        in_specs=[pl.BlockSpec((1, rows, DM), lambda i: (i, 0, 0)),
                  pl.BlockSpec((DM, DFF), lambda i: (0, 0)),
                  pl.BlockSpec((DFF, DM), lambda i: (0, 0))],
        out_specs=pl.BlockSpec((1, rows, DM), lambda i: (i, 0, 0)),
        out_shape=jax.ShapeDtypeStruct(x.shape, _F32),
    )(x, w1, w2)


G_DEC = 2
TK = CTX + PRED


def _dec_layer_body(xe_ref, y_ref, wqk_ref, wv_ref, wo_ref, w1_ref, w2_ref,
                    o_ref, s_ref):
    qrows = G_DEC * PRED
    xe = xe_ref[...].reshape(G_DEC * CTX, DM).astype(_F32)
    y = y_ref[...].reshape(qrows, DM).astype(_F32)
    ln_x = _ln(xe)
    ln_y = _ln(y)
    ln_kv = jnp.concatenate(
        [jnp.concatenate([ln_x[g * CTX:(g + 1) * CTX],
                          ln_y[g * PRED:(g + 1) * PRED]], axis=0)
         for g in range(G_DEC)], axis=0)
    ln_kvb = ln_kv.astype(_BF16)
    q = _mm(ln_y.astype(_BF16), wqk_ref[...])
    kr = _mm(ln_kvb, wqk_ref[...])
    v = _mm(ln_kvb, wv_ref[...])
    nrm2 = _mm(kr * kr, _seg_ones(DM, NH, DH))
    rinv = 1.0 / (jnp.sqrt(nrm2) + 1e-8)
    k = kr * _mm(rinv, _expand_ones(NH, DM, DH))
    qb = (q * 0.125).astype(_BF16)
    kb = k.astype(_BF16)
    for g in range(G_DEC):
        qrs = slice(g * PRED, (g + 1) * PRED)
        krs = slice(g * TK, (g + 1) * TK)
        for hh in range(NH):
            cs = slice(hh * DH, (hh + 1) * DH)
            s_ref[qrs, hh * TK:(hh + 1) * TK] = _mm_t(qb[qrs, cs], kb[krs, cs])
    s = s_ref[...]
    ii = jax.lax.broadcasted_iota(jnp.int32, (qrows, NH * TK), 0) % PRED
    jj = jax.lax.broadcasted_iota(jnp.int32, (qrows, NH * TK), 1) % TK
    mask = (jj < CTX) | ((jj - CTX) <= ii)
    s = jnp.where(mask, s, -1e9)
    m = jnp.max(s, axis=-1, keepdims=True)
    e = jnp.where(mask, jnp.exp(s - m), 0.0)
    eb = e.astype(_BF16)
    denom = _mm(eb, _seg_ones(NH * TK, NH, TK).astype(_BF16))
    dinv = 1.0 / denom
    vb = v.astype(_BF16)
    row_blocks = []
    for g in range(G_DEC):
        qrs = slice(g * PRED, (g + 1) * PRED)
        krs = slice(g * TK, (g + 1) * TK)
        cols = []
        for hh in range(NH):
            o_h = _mm(eb[qrs, hh * TK:(hh + 1) * TK],
                      vb[krs, hh * DH:(hh + 1) * DH])
            cols.append(o_h * dinv[qrs, hh:hh + 1])
        row_blocks.append(jnp.concatenate(cols, axis=1))
    o = jnp.concatenate(row_blocks, axis=0)
    ya = y + _mm(o.astype(_BF16), wo_ref[...])
    h2b = _ln(ya).astype(_BF16)
    half = DFF // 2
    t1 = jax.nn.gelu(_mm(h2b, w1_ref[:, :half]))
    o1 = _mm(t1.astype(_BF16), w2_ref[:half, :])
    t2 = jax.nn.gelu(_mm(h2b, w1_ref[:, half:]))
    o2 = _mm(t2.astype(_BF16), w2_ref[half:, :])
    o_ref[...] = (ya + o1 + o2).astype(_BF16).reshape(G_DEC, PRED, DM)


def _dec_layer(xe, y, wqk, wv, wo, w1, w2):
    b = y.shape[0]
    return pl.pallas_call(
        _dec_layer_body,
        grid=(b // G_DEC,),
        in_specs=[pl.BlockSpec((G_DEC, CTX, DM), lambda i: (i, 0, 0)),
                  pl.BlockSpec((G_DEC, PRED, DM), lambda i: (i, 0, 0)),
                  pl.BlockSpec((DM, DM), lambda i: (0, 0)),
                  pl.BlockSpec((DM, DM), lambda i: (0, 0)),
                  pl.BlockSpec((DM, DM), lambda i: (0, 0)),
                  pl.BlockSpec((DM, DFF), lambda i: (0, 0)),
                  pl.BlockSpec((DFF, DM), lambda i: (0, 0))],
        out_specs=pl.BlockSpec((G_DEC, PRED, DM), lambda i: (i, 0, 0)),
        out_shape=jax.ShapeDtypeStruct(y.shape, _BF16),
        scratch_shapes=[pltpu.VMEM((G_DEC * PRED, NH * TK), _F32)],
        compiler_params=pltpu.CompilerParams(
            dimension_semantics=("parallel",)),
    )(xe, y, wqk, wv, wo, w1, w2)


def _proj_body(y_ref, w_ref, b_ref, o_ref):
    raw = _mm(y_ref[...].astype(_F32), w_ref[...]) + b_ref[...]
    col = jax.lax.broadcasted_iota(jnp.int32, raw.shape, 1)
    sp = jnp.maximum(raw, 0.0) + jnp.log1p(jnp.exp(-jnp.abs(raw)))
    o_ref[...] = jnp.where(col == 1, raw, sp) + jnp.where(col == 0, 2.0, 0.0)


def _proj(y2d, w_pad, b_pad):
    m = y2d.shape[0]
    return pl.pallas_call(
        _proj_body,
        grid=(1,),
        in_specs=[pl.BlockSpec((m, DM), lambda i: (0, 0)),
                  pl.BlockSpec((DM, 128), lambda i: (0, 0)),
                  pl.BlockSpec((1, 128), lambda i: (0, 0))],
        out_specs=pl.BlockSpec((m, 128), lambda i: (0, 0)),
        out_shape=jax.ShapeDtypeStruct((m, 128), _F32),
    )(y2d, w_pad, b_pad)


def kernel(feat_static_cat, feat_static_real, past_time_feat, past_target,
           past_observed_values, future_time_feat, future_target,
           emb_table, W_embed, enc_Wqk, enc_Wv, enc_Wo, enc_W1, enc_W2,
           dec_Wqk, dec_Wv, dec_Wo, dec_W1, dec_W2, W_proj, b_proj):
    bsz = past_target.shape[0]
    ctx = past_target[:, -CTX:]
    obs = past_observed_values[:, -CTX:]
    denom = jnp.clip(jnp.sum(obs, axis=1, keepdims=True), 1.0, None)
    loc = jnp.sum(ctx * obs, axis=1, keepdims=True) / denom
    var = jnp.sum(((ctx - loc) * obs) ** 2, axis=1, keepdims=True) / denom
    scale = jnp.sqrt(var + 1e-5)
    inputs = (jnp.concatenate([past_target, future_target], axis=1) - loc) / scale
    time_feat = jnp.concatenate(
        [past_time_feat[:, HIST - CTX:], future_time_feat], axis=1)
    emb = emb_table[feat_static_cat[:, 0]]
    log_abs_loc = jnp.sign(loc) * jnp.log1p(jnp.abs(loc))
    log_scale = jnp.log(scale)
    static = jnp.concatenate([emb, feat_static_real, log_abs_loc, log_scale],
                             axis=1)
    feats = jnp.concatenate([
        jnp.broadcast_to(static[:, None, :], (bsz, SUB, static.shape[-1])),
        time_feat], axis=-1)
    t_len = inputs.shape[1]
    lagged = jnp.stack(
        [inputs[:, t_len - l - SUB: t_len - l] for l in LAGS], axis=-1)
    ti_in = jnp.concatenate([lagged, feats], axis=-1)
    nin = ti_in.shape[-1]
    ti_pad = jnp.pad(ti_in, ((0, 0), (0, 0), (0, 128 - nin)))
    w_pad = jnp.pad(W_embed, ((0, 128 - nin), (0, 0)))
    ti = _embed(ti_pad.reshape(bsz * SUB, 128),
                w_pad.astype(_BF16)).reshape(bsz, SUB, DM)

    x = ti[:, :CTX]
    y = ti[:, CTX:]
    enc_Wqk, enc_Wv, enc_Wo, enc_W1, enc_W2, dec_Wqk, dec_Wv, dec_Wo, \
        dec_W1, dec_W2 = (w.astype(_BF16) for w in (
            enc_Wqk, enc_Wv, enc_Wo, enc_W1, enc_W2,
            dec_Wqk, dec_Wv, dec_Wo, dec_W1, dec_W2))
    for l in range(NE):
        x = _enc_layer(x, enc_Wqk[l], enc_Wv[l], enc_Wo[l],
                       enc_W1[l], enc_W2[l])
    for l in range(ND):
        y = _dec_layer(x, y, dec_Wqk[l], dec_Wv[l], dec_Wo[l],
                       dec_W1[l], dec_W2[l])

    wp_pad = jnp.pad(W_proj, ((0, 0), (0, 128 - W_proj.shape[-1])))
    bp_pad = jnp.pad(b_proj, ((0, 128 - b_proj.shape[0]),)).reshape(1, 128)
    out = _proj(y.reshape(bsz * PRED, DM), wp_pad, bp_pad)
    return out.reshape(bsz, PRED, 128)[:, :, :3]


# speedup vs baseline: 1.0074x; 1.0074x over previous
import jax
import jax.numpy as jnp
from jax.experimental import pallas as pl
from jax.experimental.pallas import tpu as pltpu

CTX = 256
PRED = 64
LAGS = [1, 2, 3, 4, 5, 6, 7, 24, 48]
HIST = CTX + max(LAGS)
DM = 1024
NH = 16
DH = DM // NH
NE = 4
ND = 4
DFF = 4096
SUB = CTX + PRED

_F32 = jnp.float32


def _ln(x):
    m = jnp.mean(x, axis=-1, keepdims=True)
    v = jnp.mean((x - m) ** 2, axis=-1, keepdims=True)
    return (x - m) / jnp.sqrt(v + 1e-5)


def _mm(a, b):
    return jax.lax.dot_general(a, b, (((1,), (0,)), ((), ())),
                               preferred_element_type=_F32)


def _mm_t(a, b):
    return jax.lax.dot_general(a, b, (((1,), (1,)), ((), ())),
                               preferred_element_type=_F32)


_BF16 = jnp.bfloat16


def _mmb(a, b):
    return _mm(a.astype(_BF16), b.astype(_BF16))


def _mmb_t(a, b):
    return _mm_t(a.astype(_BF16), b.astype(_BF16))


def _softmax(s):
    s = s - jnp.max(s, axis=-1, keepdims=True)
    e = jnp.exp(s)
    return e / jnp.sum(e, axis=-1, keepdims=True)


def _embed_body(t_ref, w_ref, o_ref):
    o_ref[...] = _mmb(t_ref[...], w_ref[...]).astype(_BF16)


def _embed(ti_in, w_pad):
    m = ti_in.shape[0]
    blk = 2048
    return pl.pallas_call(
        _embed_body,
        grid=(m // blk,),
        in_specs=[pl.BlockSpec((blk, 128), lambda i: (i, 0)),
                  pl.BlockSpec((128, DM), lambda i: (0, 0))],
        out_specs=pl.BlockSpec((blk, DM), lambda i: (i, 0)),
        out_shape=jax.ShapeDtypeStruct((m, DM), _BF16),
    )(ti_in, w_pad)


G_ENC = 2


def _seg_ones(rows, groups, group):
    rr = jax.lax.broadcasted_iota(jnp.int32, (rows, groups), 0)
    cc = jax.lax.broadcasted_iota(jnp.int32, (rows, groups), 1)
    return (rr // group == cc).astype(_F32)


def _expand_ones(groups, cols, group):
    rr = jax.lax.broadcasted_iota(jnp.int32, (groups, cols), 0)
    cc = jax.lax.broadcasted_iota(jnp.int32, (groups, cols), 1)
    return (cc // group == rr).astype(_F32)


def _enc_layer_body(x_ref, wqkv_ref, wo_ref, w1_ref, w2_ref,
                    o_ref, s_ref):
    rows = G_ENC * CTX
    x = x_ref[...].reshape(rows, DM).astype(_F32)
    h = _ln(x)
    hb = h.astype(_BF16)
    qkv = _mm(hb, wqkv_ref[...])
    qk = qkv[:, :DM]
    v = qkv[:, DM:]
    nrm2 = _mm(qk * qk, _seg_ones(DM, NH, DH))
    rinv = 1.0 / (jnp.sqrt(nrm2) + 1e-8)
    k = qk * _mm(rinv, _expand_ones(NH, DM, DH))
    qb = (qk * 0.125).astype(_BF16)
    kb = k.astype(_BF16)
    for g in range(G_ENC):
        rs = slice(g * CTX, (g + 1) * CTX)
        for hh in range(NH):
            cs = slice(hh * DH, (hh + 1) * DH)
            s_ref[rs, hh * CTX:(hh + 1) * CTX] = _mm_t(qb[rs, cs], kb[rs, cs])
    s = s_ref[...]
    m = jnp.max(s, axis=-1, keepdims=True)
    e = jnp.exp(s - m)
    eb = e.astype(_BF16)
    denom = _mm(eb, _seg_ones(NH * CTX, NH, CTX).astype(_BF16))
    dinv = 1.0 / denom
    vb = v.astype(_BF16)
    row_blocks = []
    for g in range(G_ENC):
        rs = slice(g * CTX, (g + 1) * CTX)
        cols = []
        for hh in range(NH):
            o_h = _mm(eb[rs, hh * CTX:(hh + 1) * CTX],
                      vb[rs, hh * DH:(hh + 1) * DH])
            cols.append(o_h * dinv[rs, hh:hh + 1])
        row_blocks.append(jnp.concatenate(cols, axis=1))
    o = jnp.concatenate(row_blocks, axis=0)
    xa = x + _mm(o.astype(_BF16), wo_ref[...])
    h2b = _ln(xa).astype(_BF16)
    half = DFF // 2
    t1 = jax.nn.gelu(_mm(h2b, w1_ref[:, :half]))
    o1 = _mm(t1.astype(_BF16), w2_ref[:half, :])
    t2 = jax.nn.gelu(_mm(h2b, w1_ref[:, half:]))
    o2 = _mm(t2.astype(_BF16), w2_ref[half:, :])
    o_ref[...] = (xa + o1 + o2).astype(_BF16).reshape(G_ENC, CTX, DM)


def _enc_layer(x, wqkv, wo, w1, w2):
    b = x.shape[0]
    return pl.pallas_call(
        _enc_layer_body,
        grid=(b // G_ENC,),
        in_specs=[pl.BlockSpec((G_ENC, CTX, DM), lambda i: (i, 0, 0)),
                  pl.BlockSpec((DM, 2 * DM), lambda i: (0, 0)),
                  pl.BlockSpec((DM, DM), lambda i: (0, 0)),
                  pl.BlockSpec((DM, DFF), lambda i: (0, 0)),
                  pl.BlockSpec((DFF, DM), lambda i: (0, 0))],
        out_specs=pl.BlockSpec((G_ENC, CTX, DM), lambda i: (i, 0, 0)),
        out_shape=jax.ShapeDtypeStruct(x.shape, _BF16),
        scratch_shapes=[pltpu.VMEM((G_ENC * CTX, NH * CTX), _F32)],
    )(x, wqkv, wo, w1, w2)


def _ff_body(x_ref, w1_ref, w2_ref, o_ref):
    x = x_ref[0]
    h = _ln(x)
    t = jax.nn.gelu(_mm(h.astype(_BF16), w1_ref[...]))
    o_ref[0] = x + _mmb(t, w2_ref[...])


def _ff(x, w1, w2, rows):
    b = x.shape[0]
    return pl.pallas_call(
        _ff_body,
        grid=(b,),
        in_specs=[pl.BlockSpec((1, rows, DM), lambda i: (i, 0, 0)),
                  pl.BlockSpec((DM, DFF), lambda i: (0, 0)),
                  pl.BlockSpec((DFF, DM), lambda i: (0, 0))],
        out_specs=pl.BlockSpec((1, rows, DM), lambda i: (i, 0, 0)),
        out_shape=jax.ShapeDtypeStruct(x.shape, _F32),
    )(x, w1, w2)


G_DEC = 2
TK = CTX + PRED


def _dec_layer_body(xe_ref, y_ref, wqkv_ref, wo_ref, w1_ref, w2_ref,
                    o_ref, s_ref):
    qrows = G_DEC * PRED
    xe = xe_ref[...].reshape(G_DEC * CTX, DM).astype(_F32)
    y = y_ref[...].reshape(qrows, DM).astype(_F32)
    ln_x = _ln(xe)
    ln_y = _ln(y)
    ln_kv = jnp.concatenate(
        [jnp.concatenate([ln_x[g * CTX:(g + 1) * CTX],
                          ln_y[g * PRED:(g + 1) * PRED]], axis=0)
         for g in range(G_DEC)], axis=0)
    ln_kvb = ln_kv.astype(_BF16)
    krv = _mm(ln_kvb, wqkv_ref[...])
    kr = krv[:, :DM]
    v = krv[:, DM:]
    q = jnp.concatenate(
        [kr[g * TK + CTX:(g + 1) * TK] for g in range(G_DEC)], axis=0)
    nrm2 = _mm(kr * kr, _seg_ones(DM, NH, DH))
    rinv = 1.0 / (jnp.sqrt(nrm2) + 1e-8)
    k = kr * _mm(rinv, _expand_ones(NH, DM, DH))
    qb = (q * 0.125).astype(_BF16)
    kb = k.astype(_BF16)
    for g in range(G_DEC):
        qrs = slice(g * PRED, (g + 1) * PRED)
        krs = slice(g * TK, (g + 1) * TK)
        for hh in range(NH):
            cs = slice(hh * DH, (hh + 1) * DH)
            s_ref[qrs, hh * TK:(hh + 1) * TK] = _mm_t(qb[qrs, cs], kb[krs, cs])
    s = s_ref[...]
    ii = jax.lax.broadcasted_iota(jnp.int32, (qrows, NH * TK), 0) % PRED
    jj = jax.lax.broadcasted_iota(jnp.int32, (qrows, NH * TK), 1) % TK
    mask = (jj < CTX) | ((jj - CTX) <= ii)
    s = jnp.where(mask, s, -1e9)
    m = jnp.max(s, axis=-1, keepdims=True)
    e = jnp.where(mask, jnp.exp(s - m), 0.0)
    eb = e.astype(_BF16)
    denom = _mm(eb, _seg_ones(NH * TK, NH, TK).astype(_BF16))
    dinv = 1.0 / denom
    vb = v.astype(_BF16)
    row_blocks = []
    for g in range(G_DEC):
        qrs = slice(g * PRED, (g + 1) * PRED)
        krs = slice(g * TK, (g + 1) * TK)
        cols = []
        for hh in range(NH):
            o_h = _mm(eb[qrs, hh * TK:(hh + 1) * TK],
                      vb[krs, hh * DH:(hh + 1) * DH])
            cols.append(o_h * dinv[qrs, hh:hh + 1])
        row_blocks.append(jnp.concatenate(cols, axis=1))
    o = jnp.concatenate(row_blocks, axis=0)
    ya = y + _mm(o.astype(_BF16), wo_ref[...])
    h2b = _ln(ya).astype(_BF16)
    half = DFF // 2
    t1 = jax.nn.gelu(_mm(h2b, w1_ref[:, :half]))
    o1 = _mm(t1.astype(_BF16), w2_ref[:half, :])
    t2 = jax.nn.gelu(_mm(h2b, w1_ref[:, half:]))
    o2 = _mm(t2.astype(_BF16), w2_ref[half:, :])
    o_ref[...] = (ya + o1 + o2).astype(_BF16).reshape(G_DEC, PRED, DM)


def _dec_layer(xe, y, wqkv, wo, w1, w2):
    b = y.shape[0]
    return pl.pallas_call(
        _dec_layer_body,
        grid=(b // G_DEC,),
        in_specs=[pl.BlockSpec((G_DEC, CTX, DM), lambda i: (i, 0, 0)),
                  pl.BlockSpec((G_DEC, PRED, DM), lambda i: (i, 0, 0)),
                  pl.BlockSpec((DM, 2 * DM), lambda i: (0, 0)),
                  pl.BlockSpec((DM, DM), lambda i: (0, 0)),
                  pl.BlockSpec((DM, DFF), lambda i: (0, 0)),
                  pl.BlockSpec((DFF, DM), lambda i: (0, 0))],
        out_specs=pl.BlockSpec((G_DEC, PRED, DM), lambda i: (i, 0, 0)),
        out_shape=jax.ShapeDtypeStruct(y.shape, _BF16),
        scratch_shapes=[pltpu.VMEM((G_DEC * PRED, NH * TK), _F32)],
    )(xe, y, wqkv, wo, w1, w2)


def _proj_body(y_ref, w_ref, b_ref, o_ref):
    raw = _mm(y_ref[...].astype(_F32), w_ref[...]) + b_ref[...]
    col = jax.lax.broadcasted_iota(jnp.int32, raw.shape, 1)
    sp = jnp.maximum(raw, 0.0) + jnp.log1p(jnp.exp(-jnp.abs(raw)))
    o_ref[...] = jnp.where(col == 1, raw, sp) + jnp.where(col == 0, 2.0, 0.0)


def _proj(y2d, w_pad, b_pad):
    m = y2d.shape[0]
    return pl.pallas_call(
        _proj_body,
        grid=(1,),
        in_specs=[pl.BlockSpec((m, DM), lambda i: (0, 0)),
                  pl.BlockSpec((DM, 128), lambda i: (0, 0)),
                  pl.BlockSpec((1, 128), lambda i: (0, 0))],
        out_specs=pl.BlockSpec((m, 128), lambda i: (0, 0)),
        out_shape=jax.ShapeDtypeStruct((m, 128), _F32),
    )(y2d, w_pad, b_pad)


def kernel(feat_static_cat, feat_static_real, past_time_feat, past_target,
           past_observed_values, future_time_feat, future_target,
           emb_table, W_embed, enc_Wqk, enc_Wv, enc_Wo, enc_W1, enc_W2,
           dec_Wqk, dec_Wv, dec_Wo, dec_W1, dec_W2, W_proj, b_proj):
    bsz = past_target.shape[0]
    ctx = past_target[:, -CTX:]
    obs = past_observed_values[:, -CTX:]
    denom = jnp.clip(jnp.sum(obs, axis=1, keepdims=True), 1.0, None)
    loc = jnp.sum(ctx * obs, axis=1, keepdims=True) / denom
    var = jnp.sum(((ctx - loc) * obs) ** 2, axis=1, keepdims=True) / denom
    scale = jnp.sqrt(var + 1e-5)
    inputs = (jnp.concatenate([past_target, future_target], axis=1) - loc) / scale
    time_feat = jnp.concatenate(
        [past_time_feat[:, HIST - CTX:], future_time_feat], axis=1)
    emb = emb_table[feat_static_cat[:, 0]]
    log_abs_loc = jnp.sign(loc) * jnp.log1p(jnp.abs(loc))
    log_scale = jnp.log(scale)
    static = jnp.concatenate([emb, feat_static_real, log_abs_loc, log_scale],
                             axis=1)
    feats = jnp.concatenate([
        jnp.broadcast_to(static[:, None, :], (bsz, SUB, static.shape[-1])),
        time_feat], axis=-1)
    t_len = inputs.shape[1]
    lagged = jnp.stack(
        [inputs[:, t_len - l - SUB: t_len - l] for l in LAGS], axis=-1)
    ti_in = jnp.concatenate([lagged, feats], axis=-1)
    nin = ti_in.shape[-1]
    ti_pad = jnp.pad(ti_in, ((0, 0), (0, 0), (0, 128 - nin)))
    w_pad = jnp.pad(W_embed, ((0, 128 - nin), (0, 0)))
    ti = _embed(ti_pad.reshape(bsz * SUB, 128),
                w_pad.astype(_BF16)).reshape(bsz, SUB, DM)

    x = ti[:, :CTX]
    y = ti[:, CTX:]
    enc_Wqk, enc_Wv, enc_Wo, enc_W1, enc_W2, dec_Wqk, dec_Wv, dec_Wo, \
        dec_W1, dec_W2 = (w.astype(_BF16) for w in (
            enc_Wqk, enc_Wv, enc_Wo, enc_W1, enc_W2,
            dec_Wqk, dec_Wv, dec_Wo, dec_W1, dec_W2))
    enc_Wqkv = jnp.concatenate([enc_Wqk, enc_Wv], axis=2)
    dec_Wqkv = jnp.concatenate([dec_Wqk, dec_Wv], axis=2)
    for l in range(NE):
        x = _enc_layer(x, enc_Wqkv[l], enc_Wo[l], enc_W1[l], enc_W2[l])
    for l in range(ND):
        y = _dec_layer(x, y, dec_Wqkv[l], dec_Wo[l], dec_W1[l], dec_W2[l])

    wp_pad = jnp.pad(W_proj, ((0, 0), (0, 128 - W_proj.shape[-1])))
    bp_pad = jnp.pad(b_proj, ((0, 128 - b_proj.shape[0]),)).reshape(1, 128)
    out = _proj(y.reshape(bsz * PRED, DM), wp_pad, bp_pad)
    return out.reshape(bsz, PRED, 128)[:, :, :3]


# speedup vs baseline: 1.0157x; 1.0083x over previous
import jax
import jax.numpy as jnp
from jax.experimental import pallas as pl
from jax.experimental.pallas import tpu as pltpu

CTX = 256
PRED = 64
LAGS = [1, 2, 3, 4, 5, 6, 7, 24, 48]
HIST = CTX + max(LAGS)
DM = 1024
NH = 16
DH = DM // NH
NE = 4
ND = 4
DFF = 4096
SUB = CTX + PRED

_F32 = jnp.float32


def _ln(x):
    m = jnp.mean(x, axis=-1, keepdims=True)
    v = jnp.mean((x - m) ** 2, axis=-1, keepdims=True)
    return (x - m) / jnp.sqrt(v + 1e-5)


def _mm(a, b):
    return jax.lax.dot_general(a, b, (((1,), (0,)), ((), ())),
                               preferred_element_type=_F32)


def _mm_t(a, b):
    return jax.lax.dot_general(a, b, (((1,), (1,)), ((), ())),
                               preferred_element_type=_F32)


_BF16 = jnp.bfloat16


def _mmb(a, b):
    return _mm(a.astype(_BF16), b.astype(_BF16))


def _mmb_t(a, b):
    return _mm_t(a.astype(_BF16), b.astype(_BF16))


def _softmax(s):
    s = s - jnp.max(s, axis=-1, keepdims=True)
    e = jnp.exp(s)
    return e / jnp.sum(e, axis=-1, keepdims=True)


def _embed_body(t_ref, w_ref, o_ref):
    o_ref[...] = _mmb(t_ref[...], w_ref[...]).astype(_BF16)


def _embed(ti_in, w_pad):
    m = ti_in.shape[0]
    blk = 2048
    return pl.pallas_call(
        _embed_body,
        grid=(m // blk,),
        in_specs=[pl.BlockSpec((blk, 128), lambda i: (i, 0)),
                  pl.BlockSpec((128, DM), lambda i: (0, 0))],
        out_specs=pl.BlockSpec((blk, DM), lambda i: (i, 0)),
        out_shape=jax.ShapeDtypeStruct((m, DM), _BF16),
    )(ti_in, w_pad)


G_ENC = 2


def _seg_ones(rows, groups, group):
    rr = jax.lax.broadcasted_iota(jnp.int32, (rows, groups), 0)
    cc = jax.lax.broadcasted_iota(jnp.int32, (rows, groups), 1)
    return (rr // group == cc).astype(_F32)


def _expand_ones(groups, cols, group):
    rr = jax.lax.broadcasted_iota(jnp.int32, (groups, cols), 0)
    cc = jax.lax.broadcasted_iota(jnp.int32, (groups, cols), 1)
    return (cc // group == rr).astype(_F32)


def _enc_layer_body(x_ref, wqkv_ref, wo_ref, w1_ref, w2_ref,
                    o_ref, s_ref):
    rows = G_ENC * CTX
    x = x_ref[...].reshape(rows, DM).astype(_F32)
    h = _ln(x)
    hb = h.astype(_BF16)
    qkv = _mm(hb, wqkv_ref[...])
    qk = qkv[:, :DM]
    v = qkv[:, DM:]
    nrm2 = _mm(qk * qk, _seg_ones(DM, NH, DH))
    rinv = 1.0 / (jnp.sqrt(nrm2) + 1e-8)
    k = qk * _mm(rinv, _expand_ones(NH, DM, DH))
    qb = (qk * 0.125).astype(_BF16)
    kb = k.astype(_BF16)
    for g in range(G_ENC):
        rs = slice(g * CTX, (g + 1) * CTX)
        for hh in range(NH):
            cs = slice(hh * DH, (hh + 1) * DH)
            s_ref[rs, hh * CTX:(hh + 1) * CTX] = _mm_t(qb[rs, cs], kb[rs, cs])
    s = s_ref[...]
    m = jnp.max(s, axis=-1, keepdims=True)
    e = jnp.exp(s - m)
    eb = e.astype(_BF16)
    denom = _mm(eb, _seg_ones(NH * CTX, NH, CTX).astype(_BF16))
    dinv = 1.0 / denom
    vb = v.astype(_BF16)
    row_blocks = []
    for g in range(G_ENC):
        rs = slice(g * CTX, (g + 1) * CTX)
        cols = []
        for hh in range(NH):
            o_h = _mm(eb[rs, hh * CTX:(hh + 1) * CTX],
                      vb[rs, hh * DH:(hh + 1) * DH])
            cols.append(o_h * dinv[rs, hh:hh + 1])
        row_blocks.append(jnp.concatenate(cols, axis=1))
    o = jnp.concatenate(row_blocks, axis=0)
    xa = x + _mm(o.astype(_BF16), wo_ref[...])
    h2b = _ln(xa).astype(_BF16)
    half = DFF // 2
    t1 = jax.nn.gelu(_mm(h2b, w1_ref[:, :half]))
    o1 = _mm(t1.astype(_BF16), w2_ref[:half, :])
    t2 = jax.nn.gelu(_mm(h2b, w1_ref[:, half:]))
    o2 = _mm(t2.astype(_BF16), w2_ref[half:, :])
    o_ref[...] = (xa + o1 + o2).astype(_BF16).reshape(G_ENC, CTX, DM)


def _enc_layer(x, wqkv, wo, w1, w2):
    b = x.shape[0]
    return pl.pallas_call(
        _enc_layer_body,
        grid=(b // G_ENC,),
        in_specs=[pl.BlockSpec((G_ENC, CTX, DM), lambda i: (i, 0, 0)),
                  pl.BlockSpec((DM, 2 * DM), lambda i: (0, 0)),
                  pl.BlockSpec((DM, DM), lambda i: (0, 0)),
                  pl.BlockSpec((DM, DFF), lambda i: (0, 0)),
                  pl.BlockSpec((DFF, DM), lambda i: (0, 0))],
        out_specs=pl.BlockSpec((G_ENC, CTX, DM), lambda i: (i, 0, 0)),
        out_shape=jax.ShapeDtypeStruct(x.shape, _BF16),
        scratch_shapes=[pltpu.VMEM((G_ENC * CTX, NH * CTX), _F32)],
    )(x, wqkv, wo, w1, w2)


def _ff_body(x_ref, w1_ref, w2_ref, o_ref):
    x = x_ref[0]
    h = _ln(x)
    t = jax.nn.gelu(_mm(h.astype(_BF16), w1_ref[...]))
    o_ref[0] = x + _mmb(t, w2_ref[...])


def _ff(x, w1, w2, rows):
    b = x.shape[0]
    return pl.pallas_call(
        _ff_body,
        grid=(b,),
        in_specs=[pl.BlockSpec((1, rows, DM), lambda i: (i, 0, 0)),
                  pl.BlockSpec((DM, DFF), lambda i: (0, 0)),
                  pl.BlockSpec((DFF, DM), lambda i: (0, 0))],
        out_specs=pl.BlockSpec((1, rows, DM), lambda i: (i, 0, 0)),
        out_shape=jax.ShapeDtypeStruct(x.shape, _F32),
    )(x, w1, w2)


G_DEC = 4
TK = CTX + PRED


def _dec_layer_body(xe_ref, y_ref, wqkv_ref, wo_ref, w1_ref, w2_ref,
                    o_ref, s_ref):
    qrows = G_DEC * PRED
    xe = xe_ref[...].reshape(G_DEC * CTX, DM).astype(_F32)
    y = y_ref[...].reshape(qrows, DM).astype(_F32)
    ln_x = _ln(xe)
    ln_y = _ln(y)
    ln_kv = jnp.concatenate(
        [jnp.concatenate([ln_x[g * CTX:(g + 1) * CTX],
                          ln_y[g * PRED:(g + 1) * PRED]], axis=0)
         for g in range(G_DEC)], axis=0)
    ln_kvb = ln_kv.astype(_BF16)
    krv = _mm(ln_kvb, wqkv_ref[...])
    kr = krv[:, :DM]
    v = krv[:, DM:]
    q = jnp.concatenate(
        [kr[g * TK + CTX:(g + 1) * TK] for g in range(G_DEC)], axis=0)
    nrm2 = _mm(kr * kr, _seg_ones(DM, NH, DH))
    rinv = 1.0 / (jnp.sqrt(nrm2) + 1e-8)
    k = kr * _mm(rinv, _expand_ones(NH, DM, DH))
    qb = (q * 0.125).astype(_BF16)
    kb = k.astype(_BF16)
    for g in range(G_DEC):
        qrs = slice(g * PRED, (g + 1) * PRED)
        krs = slice(g * TK, (g + 1) * TK)
        for hh in range(NH):
            cs = slice(hh * DH, (hh + 1) * DH)
            s_ref[qrs, hh * TK:(hh + 1) * TK] = _mm_t(qb[qrs, cs], kb[krs, cs])
    s = s_ref[...]
    ii = jax.lax.broadcasted_iota(jnp.int32, (qrows, NH * TK), 0) % PRED
    jj = jax.lax.broadcasted_iota(jnp.int32, (qrows, NH * TK), 1) % TK
    mask = (jj < CTX) | ((jj - CTX) <= ii)
    s = jnp.where(mask, s, -1e9)
    m = jnp.max(s, axis=-1, keepdims=True)
    e = jnp.where(mask, jnp.exp(s - m), 0.0)
    eb = e.astype(_BF16)
    denom = _mm(eb, _seg_ones(NH * TK, NH, TK).astype(_BF16))
    dinv = 1.0 / denom
    vb = v.astype(_BF16)
    row_blocks = []
    for g in range(G_DEC):
        qrs = slice(g * PRED, (g + 1) * PRED)
        krs = slice(g * TK, (g + 1) * TK)
        cols = []
        for hh in range(NH):
            o_h = _mm(eb[qrs, hh * TK:(hh + 1) * TK],
                      vb[krs, hh * DH:(hh + 1) * DH])
            cols.append(o_h * dinv[qrs, hh:hh + 1])
        row_blocks.append(jnp.concatenate(cols, axis=1))
    o = jnp.concatenate(row_blocks, axis=0)
    ya = y + _mm(o.astype(_BF16), wo_ref[...])
    h2b = _ln(ya).astype(_BF16)
    half = DFF // 2
    t1 = jax.nn.gelu(_mm(h2b, w1_ref[:, :half]))
    o1 = _mm(t1.astype(_BF16), w2_ref[:half, :])
    t2 = jax.nn.gelu(_mm(h2b, w1_ref[:, half:]))
    o2 = _mm(t2.astype(_BF16), w2_ref[half:, :])
    o_ref[...] = (ya + o1 + o2).astype(_BF16).reshape(G_DEC, PRED, DM)


def _dec_layer(xe, y, wqkv, wo, w1, w2):
    b = y.shape[0]
    return pl.pallas_call(
        _dec_layer_body,
        grid=(b // G_DEC,),
        in_specs=[pl.BlockSpec((G_DEC, CTX, DM), lambda i: (i, 0, 0)),
                  pl.BlockSpec((G_DEC, PRED, DM), lambda i: (i, 0, 0)),
                  pl.BlockSpec((DM, 2 * DM), lambda i: (0, 0)),
                  pl.BlockSpec((DM, DM), lambda i: (0, 0)),
                  pl.BlockSpec((DM, DFF), lambda i: (0, 0)),
                  pl.BlockSpec((DFF, DM), lambda i: (0, 0))],
        out_specs=pl.BlockSpec((G_DEC, PRED, DM), lambda i: (i, 0, 0)),
        out_shape=jax.ShapeDtypeStruct(y.shape, _BF16),
        scratch_shapes=[pltpu.VMEM((G_DEC * PRED, NH * TK), _F32)],
    )(xe, y, wqkv, wo, w1, w2)


def _proj_body(y_ref, w_ref, b_ref, o_ref):
    raw = _mm(y_ref[...].astype(_F32), w_ref[...]) + b_ref[...]
    col = jax.lax.broadcasted_iota(jnp.int32, raw.shape, 1)
    sp = jnp.maximum(raw, 0.0) + jnp.log1p(jnp.exp(-jnp.abs(raw)))
    o_ref[...] = jnp.where(col == 1, raw, sp) + jnp.where(col == 0, 2.0, 0.0)


def _proj(y2d, w_pad, b_pad):
    m = y2d.shape[0]
    return pl.pallas_call(
        _proj_body,
        grid=(1,),
        in_specs=[pl.BlockSpec((m, DM), lambda i: (0, 0)),
                  pl.BlockSpec((DM, 128), lambda i: (0, 0)),
                  pl.BlockSpec((1, 128), lambda i: (0, 0))],
        out_specs=pl.BlockSpec((m, 128), lambda i: (0, 0)),
        out_shape=jax.ShapeDtypeStruct((m, 128), _F32),
    )(y2d, w_pad, b_pad)


def kernel(feat_static_cat, feat_static_real, past_time_feat, past_target,
           past_observed_values, future_time_feat, future_target,
           emb_table, W_embed, enc_Wqk, enc_Wv, enc_Wo, enc_W1, enc_W2,
           dec_Wqk, dec_Wv, dec_Wo, dec_W1, dec_W2, W_proj, b_proj):
    bsz = past_target.shape[0]
    ctx = past_target[:, -CTX:]
    obs = past_observed_values[:, -CTX:]
    denom = jnp.clip(jnp.sum(obs, axis=1, keepdims=True), 1.0, None)
    loc = jnp.sum(ctx * obs, axis=1, keepdims=True) / denom
    var = jnp.sum(((ctx - loc) * obs) ** 2, axis=1, keepdims=True) / denom
    scale = jnp.sqrt(var + 1e-5)
    inputs = (jnp.concatenate([past_target, future_target], axis=1) - loc) / scale
    time_feat = jnp.concatenate(
        [past_time_feat[:, HIST - CTX:], future_time_feat], axis=1)
    emb = emb_table[feat_static_cat[:, 0]]
    log_abs_loc = jnp.sign(loc) * jnp.log1p(jnp.abs(loc))
    log_scale = jnp.log(scale)
    static = jnp.concatenate([emb, feat_static_real, log_abs_loc, log_scale],
                             axis=1)
    feats = jnp.concatenate([
        jnp.broadcast_to(static[:, None, :], (bsz, SUB, static.shape[-1])),
        time_feat], axis=-1)
    t_len = inputs.shape[1]
    lagged = jnp.stack(
        [inputs[:, t_len - l - SUB: t_len - l] for l in LAGS], axis=-1)
    ti_in = jnp.concatenate([lagged, feats], axis=-1)
    nin = ti_in.shape[-1]
    ti_pad = jnp.pad(ti_in, ((0, 0), (0, 0), (0, 128 - nin)))
    w_pad = jnp.pad(W_embed, ((0, 128 - nin), (0, 0)))
    ti = _embed(ti_pad.reshape(bsz * SUB, 128),
                w_pad.astype(_BF16)).reshape(bsz, SUB, DM)

    x = ti[:, :CTX]
    y = ti[:, CTX:]
    enc_Wqk, enc_Wv, enc_Wo, enc_W1, enc_W2, dec_Wqk, dec_Wv, dec_Wo, \
        dec_W1, dec_W2 = (w.astype(_BF16) for w in (
            enc_Wqk, enc_Wv, enc_Wo, enc_W1, enc_W2,
            dec_Wqk, dec_Wv, dec_Wo, dec_W1, dec_W2))
    enc_Wqkv = jnp.concatenate([enc_Wqk, enc_Wv], axis=2)
    dec_Wqkv = jnp.concatenate([dec_Wqk, dec_Wv], axis=2)
    for l in range(NE):
        x = _enc_layer(x, enc_Wqkv[l], enc_Wo[l], enc_W1[l], enc_W2[l])
    for l in range(ND):
        y = _dec_layer(x, y, dec_Wqkv[l], dec_Wo[l], dec_W1[l], dec_W2[l])

    wp_pad = jnp.pad(W_proj, ((0, 0), (0, 128 - W_proj.shape[-1])))
    bp_pad = jnp.pad(b_proj, ((0, 128 - b_proj.shape[0]),)).reshape(1, 128)
    out = _proj(y.reshape(bsz * PRED, DM), wp_pad, bp_pad)
    return out.reshape(bsz, PRED, 128)[:, :, :3]
